# baseline (device time: 58058 ns/iter reference)
import jax
import jax.numpy as jnp
from jax import lax
from jax.experimental import pallas as pl
from jax.experimental.pallas import tpu as pltpu

N_DEV = 16
N_LAYERS = 3
G = 4


def kernel(x, Win0, Wout0, Win1, Wout1, Win2, Wout2):
    b, d_shard = x.shape
    h_dim = Win0.shape[1]
    rows = b // N_DEV
    blk = rows * G

    def body(x_ref, win0_ref, wout0_ref, win1_ref, wout1_ref, win2_ref,
             wout2_ref, out_ref, acc_ref, s1_recv, s2_src, s2_recv, h_full,
             s1_ssem, s1_rsem, s2_ssem, s2_rsem,
             g2_ssem, g2_rsem, g1_ssem, g1_rsem):
        my_i = lax.axis_index("i")
        z_me = my_i // G
        r_me = my_i % G
        own_rows = blk * r_me + rows * z_me
        wins = [win0_ref, win1_ref, win2_ref]
        wouts = [wout0_ref, wout1_ref, wout2_ref]
        all_rdmas = []

        def remote_copy(src, dst, ssem, rsem, target):
            r = pltpu.make_async_remote_copy(
                src_ref=src, dst_ref=dst, send_sem=ssem, recv_sem=rsem,
                device_id=(target,), device_id_type=pl.DeviceIdType.MESH,
            )
            r.start()
            all_rdmas.append(r)

        def wait_recv(dst, rsem):
            w = pltpu.make_async_remote_copy(
                src_ref=dst, dst_ref=dst, send_sem=rsem, recv_sem=rsem,
                device_id=(my_i,), device_id_type=pl.DeviceIdType.MESH,
            )
            w.wait_recv()

        def rs1_sends(l):
            for k in range(1, G):
                r_t = (r_me + k) % G
                remote_copy(acc_ref.at[l, pl.ds(blk * r_t, blk), :],
                            s1_recv.at[l, G - k],
                            s1_ssem.at[l, G - k], s1_rsem.at[l, G - k],
                            G * z_me + r_t)

        acc_ref[0] = jnp.dot(
            x_ref[...], win0_ref[...], preferred_element_type=jnp.float32
        ).astype(jnp.bfloat16)
        rs1_sends(0)

        for l in range(N_LAYERS):
            s_blk = acc_ref[l, pl.ds(blk * r_me, blk), :].astype(jnp.float32)
            for k in range(1, G):
                wait_recv(s1_recv.at[l, k], s1_rsem.at[l, k])
                s_blk = s_blk + s1_recv[l, k].astype(jnp.float32)
            s2_src[l] = s_blk.astype(jnp.bfloat16)

            for k in range(1, G):
                z_t = (z_me + k) % G
                remote_copy(s2_src.at[l, pl.ds(rows * z_t, rows), :],
                            s2_recv.at[l, G - k],
                            s2_ssem.at[l, G - k], s2_rsem.at[l, G - k],
                            G * z_t + r_me)
            red = s2_src[l, pl.ds(rows * z_me, rows), :].astype(jnp.float32)
            for k in range(1, G):
                wait_recv(s2_recv.at[l, k], s2_rsem.at[l, k])
                red = red + s2_recv[l, k].astype(jnp.float32)
            h_chunk = jnp.maximum(red, 0.0)
            h_full[l, pl.ds(own_rows, rows), :] = h_chunk.astype(jnp.bfloat16)

            for k in range(1, G):
                z_t = (z_me + k) % G
                remote_copy(h_full.at[l, pl.ds(own_rows, rows), :],
                            h_full.at[l, pl.ds(own_rows, rows), :],
                            g2_ssem.at[l, G - k], g2_rsem.at[l, G - k],
                            G * z_t + r_me)
            for k in range(1, G):
                wait_recv(h_full.at[l, pl.ds(0, rows), :], g2_rsem.at[l, k])

            for k in range(1, G):
                r_t = (r_me + k) % G
                remote_copy(h_full.at[l, pl.ds(blk * r_me, blk), :],
                            h_full.at[l, pl.ds(blk * r_me, blk), :],
                            g1_ssem.at[l, G - k], g1_rsem.at[l, G - k],
                            G * z_me + r_t)
            for k in range(1, G):
                wait_recv(h_full.at[l, pl.ds(0, blk), :], g1_rsem.at[l, k])

            y = jnp.dot(
                h_full[l], wouts[l][...], preferred_element_type=jnp.float32
            )
            if l < N_LAYERS - 1:
                acc_ref[l + 1] = jnp.dot(
                    y, wins[l + 1][...], preferred_element_type=jnp.float32
                ).astype(jnp.bfloat16)
                rs1_sends(l + 1)
            else:
                out_ref[...] = y

        for r in all_rdmas:
            r.wait_send()

    return pl.pallas_call(
        body,
        out_shape=jax.ShapeDtypeStruct((b, d_shard), jnp.float32),
        in_specs=[pl.BlockSpec(memory_space=pltpu.VMEM)] * 7,
        out_specs=pl.BlockSpec(memory_space=pltpu.VMEM),
        scratch_shapes=[
            pltpu.VMEM((N_LAYERS, b, h_dim), jnp.bfloat16),
            pltpu.VMEM((N_LAYERS, G, blk, h_dim), jnp.bfloat16),
            pltpu.VMEM((N_LAYERS, blk, h_dim), jnp.bfloat16),
            pltpu.VMEM((N_LAYERS, G, rows, h_dim), jnp.bfloat16),
            pltpu.VMEM((N_LAYERS, b, h_dim), jnp.bfloat16),
            pltpu.SemaphoreType.DMA((N_LAYERS, G)),
            pltpu.SemaphoreType.DMA((N_LAYERS, G)),
            pltpu.SemaphoreType.DMA((N_LAYERS, G)),
            pltpu.SemaphoreType.DMA((N_LAYERS, G)),
            pltpu.SemaphoreType.DMA((N_LAYERS, G)),
            pltpu.SemaphoreType.DMA((N_LAYERS, G)),
            pltpu.SemaphoreType.DMA((N_LAYERS, G)),
            pltpu.SemaphoreType.DMA((N_LAYERS, G)),
        ],
    )(x, Win0, Wout0, Win1, Wout1, Win2, Wout2)


# device time: 49349 ns/iter; 1.1765x vs baseline; 1.1765x over previous
import jax
import jax.numpy as jnp
from jax import lax
from jax.experimental import pallas as pl
from jax.experimental.pallas import tpu as pltpu

N_DEV = 16
N_PEERS = N_DEV - 1
N_LAYERS = 3
GROUPS = ((0, 8), (8, 16))


def kernel(x, Win0, Wout0, Win1, Wout1, Win2, Wout2):
    b, d_shard = x.shape
    h_dim = Win0.shape[1]
    rows = b // N_DEV

    def body(x_ref, win0_ref, wout0_ref, win1_ref, wout1_ref, win2_ref,
             wout2_ref, out_ref, acc_ref, rs_recv, h_slot,
             rs_ssem, rs_rsem, ag_ssem, ag_rsem):
        my_i = lax.axis_index("i")
        wins = [win0_ref, win1_ref, win2_ref]
        wouts = [wout0_ref, wout1_ref, wout2_ref]
        all_rdmas = []

        def rs_send_l0(j):
            p = (my_i + 1 + j) % N_DEV
            r = pltpu.make_async_remote_copy(
                src_ref=acc_ref.at[0, pl.ds(rows * p, rows), :],
                dst_ref=rs_recv.at[0, j],
                send_sem=rs_ssem.at[0, j],
                recv_sem=rs_rsem.at[0, j],
                device_id=(p,),
                device_id_type=pl.DeviceIdType.MESH,
            )
            r.start()
            all_rdmas.append(r)

        def rs_send(l, j):
            p = (my_i - 1 - j) % N_DEV
            r = pltpu.make_async_remote_copy(
                src_ref=acc_ref.at[l, pl.ds(rows * j, rows), :],
                dst_ref=rs_recv.at[l, 14 - j],
                send_sem=rs_ssem.at[l, j],
                recv_sem=rs_rsem.at[l, 14 - j],
                device_id=(p,),
                device_id_type=pl.DeviceIdType.MESH,
            )
            r.start()
            all_rdmas.append(r)

        acc_ref[0] = jnp.dot(
            x_ref[...], win0_ref[...], preferred_element_type=jnp.float32
        ).astype(jnp.bfloat16)
        for j in range(N_PEERS):
            rs_send_l0(j)

        own_f32 = None
        for l in range(N_LAYERS):
            if l == 0:
                red = acc_ref[0, pl.ds(rows * my_i, rows), :].astype(
                    jnp.float32
                )
            else:
                red = own_f32
            rs_waits = []
            for j in range(N_PEERS):
                w = pltpu.make_async_remote_copy(
                    src_ref=acc_ref.at[l, pl.ds(0, rows), :],
                    dst_ref=rs_recv.at[l, j],
                    send_sem=rs_ssem.at[l, j],
                    recv_sem=rs_rsem.at[l, j],
                    device_id=(my_i,),
                    device_id_type=pl.DeviceIdType.MESH,
                )
                w.wait_recv()
                red = red + rs_recv[l, j].astype(jnp.float32)
            h_chunk = jnp.maximum(red, 0.0)
            h_slot[l, N_DEV - 1] = h_chunk.astype(jnp.bfloat16)

            ag_rdmas = []
            for j in range(N_PEERS):
                p = (my_i + 1 + j) % N_DEV
                r = pltpu.make_async_remote_copy(
                    src_ref=h_slot.at[l, N_DEV - 1],
                    dst_ref=h_slot.at[l, j],
                    send_sem=ag_ssem.at[l, j],
                    recv_sem=ag_rsem.at[l, j],
                    device_id=(p,),
                    device_id_type=pl.DeviceIdType.MESH,
                )
                r.start()
                ag_rdmas.append(r)
                all_rdmas.append(r)

            for lo, hi in GROUPS:
                for j in range(lo, min(hi, N_PEERS)):
                    ag_rdmas[j].wait_recv()
                hh = h_slot[l, lo:hi].reshape((hi - lo) * rows, h_dim)
                y = jnp.dot(
                    hh, wouts[l][...], preferred_element_type=jnp.float32
                )
                if l < N_LAYERS - 1:
                    pa = jnp.dot(
                        y, wins[l + 1][...],
                        preferred_element_type=jnp.float32,
                    )
                    acc_ref[l + 1, pl.ds(rows * lo, (hi - lo) * rows), :] = (
                        pa.astype(jnp.bfloat16)
                    )
                    for j in range(lo, min(hi, N_PEERS)):
                        rs_send(l + 1, j)
                    if hi == N_DEV:
                        own_f32 = pa[(N_DEV - 1 - lo) * rows:, :]
                else:
                    for j in range(lo, min(hi, N_PEERS)):
                        i_org = (my_i - 1 - j) % N_DEV
                        out_ref[pl.ds(rows * i_org, rows), :] = y[
                            (j - lo) * rows:(j - lo + 1) * rows, :
                        ]
                    if hi == N_DEV:
                        out_ref[pl.ds(rows * my_i, rows), :] = y[
                            (N_DEV - 1 - lo) * rows:, :
                        ]

        for r in all_rdmas:
            r.wait_send()

    return pl.pallas_call(
        body,
        out_shape=jax.ShapeDtypeStruct((b, d_shard), jnp.float32),
        in_specs=[pl.BlockSpec(memory_space=pltpu.VMEM)] * 7,
        out_specs=pl.BlockSpec(memory_space=pltpu.VMEM),
        scratch_shapes=[
            pltpu.VMEM((N_LAYERS, b, h_dim), jnp.bfloat16),
            pltpu.VMEM((N_LAYERS, N_PEERS, rows, h_dim), jnp.bfloat16),
            pltpu.VMEM((N_LAYERS, N_DEV, rows, h_dim), jnp.bfloat16),
            pltpu.SemaphoreType.DMA((N_LAYERS, N_PEERS)),
            pltpu.SemaphoreType.DMA((N_LAYERS, N_PEERS)),
            pltpu.SemaphoreType.DMA((N_LAYERS, N_PEERS)),
            pltpu.SemaphoreType.DMA((N_LAYERS, N_PEERS)),
        ],
    )(x, Win0, Wout0, Win1, Wout1, Win2, Wout2)
